# baseline (speedup 1.0000x reference)
import functools

import jax
import jax.numpy as jnp
from jax import lax
from jax.experimental import pallas as pl
from jax.experimental.pallas import tpu as pltpu
from jax.experimental.pallas import tpu_sc as plsc

N_NODES = 10000
N_EDGES = 320000
D = 128
EPS_BN = 1e-5

NC = 2
NS = 16
NW = NC * NS
EDGES_PER_WORKER = N_EDGES // NW
CHUNK = 80
NCHUNKS = 128
EW_PAD = NCHUNKS * CHUNK
PASS = 64
IDX_DST = 64
N_PAD = 10240
ROWS_PER_TILE = N_PAD // NS
PAD_ROW = N_NODES


def _sc_agg_body(with_deg, *refs):
    if with_deg:
        (h_hbm, src_hbm, dst_hbm, out_hbm, deg_hbm,
         idx_all, rows_a, rows_b, acc_sh, sem_a, sem_b) = refs
    else:
        (h_hbm, src_hbm, dst_hbm, out_hbm,
         idx_all, rows_a, rows_b, acc_sh, sem_a, sem_b) = refs

    c = lax.axis_index("c")
    s = lax.axis_index("s")
    w = c * NS + s
    row0 = s * ROWS_PER_TILE

    def stage_src(p):
        pltpu.sync_copy(src_hbm.at[w, pl.ds(p * PASS, PASS)],
                        idx_all.at[pl.ds(0, PASS)])

    def stage_dst(p):
        pltpu.sync_copy(dst_hbm.at[w, pl.ds(p * PASS, PASS)],
                        idx_all.at[pl.ds(IDX_DST, PASS)])

    zero16 = jnp.zeros((16,), jnp.float32)
    for r in range(CHUNK):
        for q in range(D // 16):
            rows_b[r, pl.ds(q * 16, 16)] = zero16

    def zero_acc():
        for i in range(ROWS_PER_TILE // CHUNK):
            pltpu.sync_copy(rows_b, acc_sh.at[pl.ds(row0 + i * CHUNK, CHUNK)])

    if with_deg:
        one16 = jnp.ones((16,), jnp.float32)
        for r in range(CHUNK):
            for q in range(D // 16):
                rows_a[r, pl.ds(q * 16, 16)] = one16
        zero_acc()
        plsc.subcore_barrier()

        for p in range(2):
            stage_dst(p)

            @pl.loop(0, PASS)
            def _(j):
                pltpu.sync_copy(rows_a, acc_sh.at[idx_all.at[IDX_DST + j]],
                                add=True)

        plsc.subcore_barrier()
        pltpu.sync_copy(acc_sh.at[pl.ds(row0, ROWS_PER_TILE)],
                        deg_hbm.at[c, pl.ds(row0, ROWS_PER_TILE)])

    zero_acc()
    plsc.subcore_barrier()

    def gather(j, buf, sem):
        return pltpu.async_copy(h_hbm.at[idx_all.at[j]], buf, sem)

    def gather_wait(j, buf, sem):
        pltpu.make_async_copy(h_hbm.at[idx_all.at[j]], buf, sem).wait()

    def scatter(j, buf):
        pltpu.sync_copy(buf, acc_sh.at[idx_all.at[IDX_DST + j]], add=True)

    for p in range(2):
        stage_src(p)
        stage_dst(p)

        @pl.loop(0, PASS)
        def _(j):
            gather(j, rows_a, sem_a).wait()
            scatter(j, rows_a)

    plsc.subcore_barrier()

    pltpu.sync_copy(acc_sh.at[pl.ds(row0, ROWS_PER_TILE)],
                    out_hbm.at[c, pl.ds(row0, ROWS_PER_TILE)])


def _make_sc_agg(with_deg):
    mesh = plsc.VectorSubcoreMesh(core_axis_name="c", subcore_axis_name="s",
                                  num_cores=NC, num_subcores=NS)
    out_type = [jax.ShapeDtypeStruct((NC, N_PAD, D), jnp.float32)]
    if with_deg:
        out_type.append(jax.ShapeDtypeStruct((NC, N_PAD, D), jnp.float32))
    scratch = [
        pltpu.VMEM((2 * PASS, CHUNK), jnp.int32),
        pltpu.VMEM((CHUNK, D), jnp.float32),
        pltpu.VMEM((CHUNK, D), jnp.float32),
        pltpu.VMEM_SHARED((N_PAD, D), jnp.float32),
        pltpu.SemaphoreType.DMA,
        pltpu.SemaphoreType.DMA,
    ]
    return pl.kernel(
        functools.partial(_sc_agg_body, with_deg),
        out_type=out_type,
        mesh=mesh,
        scratch_types=scratch,
    )


def _tc_layer_body(first, bn, p_ref, h_ref, aux_ref, w_ref, b_ref,
                   gamma_ref, beta_ref, o_ref, inv_ref=None):
    if first:
        deg = aux_ref[0, :N_NODES, 0:1] + aux_ref[1, :N_NODES, 0:1]
        inv = 1.0 / jnp.maximum(deg, 1.0)
        inv_ref[...] = inv
    else:
        inv = aux_ref[...]
    p = p_ref[0, :N_NODES, :D] + p_ref[1, :N_NODES, :D]
    t = h_ref[...] + p * inv
    y = lax.dot_general(t, w_ref[...], (((1,), (1,)), ((), ())),
                        preferred_element_type=jnp.float32) + b_ref[...]
    if bn:
        mu = jnp.mean(y, axis=0, keepdims=True)
        var = jnp.mean((y - mu) * (y - mu), axis=0, keepdims=True)
        y = gamma_ref[...] * (y - mu) * lax.rsqrt(var + EPS_BN) + beta_ref[...]
        y = jnp.maximum(y, 0.0)
    o_ref[...] = y


def _make_tc_layer(first, bn):
    out_shape = [jax.ShapeDtypeStruct((N_NODES, D), jnp.float32)]
    if first:
        out_shape.append(jax.ShapeDtypeStruct((N_NODES, 1), jnp.float32))
    return pl.pallas_call(
        functools.partial(_tc_layer_body, first, bn),
        out_shape=out_shape,
    )


def kernel(x, edge_index, W0, b0, W1, b1, W2, b2,
           gamma0, beta0, gamma1, beta1):
    src = edge_index[0].astype(jnp.int32).reshape(NW, EDGES_PER_WORKER)
    dst = edge_index[1].astype(jnp.int32).reshape(NW, EDGES_PER_WORKER)
    pad = EW_PAD - EDGES_PER_WORKER
    src = jnp.concatenate(
        [src, jnp.zeros((NW, pad), jnp.int32)], axis=1
    ).reshape(NW, NCHUNKS, CHUNK)
    pad_rows = PAD_ROW + jnp.arange(pad, dtype=jnp.int32) % (N_PAD - N_NODES)
    dst = jnp.concatenate(
        [dst, jnp.broadcast_to(pad_rows, (NW, pad))], axis=1
    ).reshape(NW, NCHUNKS, CHUNK)

    sc_agg_deg = _make_sc_agg(True)
    sc_agg = _make_sc_agg(False)
    tc_first = _make_tc_layer(True, True)
    tc_mid = _make_tc_layer(False, True)
    tc_last = _make_tc_layer(False, False)

    b0r = b0.reshape(1, D)
    b1r = b1.reshape(1, D)
    b2r = b2.reshape(1, D)
    g0 = gamma0.reshape(1, D)
    g1 = gamma1.reshape(1, D)
    be0 = beta0.reshape(1, D)
    be1 = beta1.reshape(1, D)

    p1, degp = sc_agg_deg(x, src, dst)
    h1, inv = tc_first(p1, x, degp, W0, b0r, g0, be0)
    (p2,) = sc_agg(h1, src, dst)
    (h2,) = tc_mid(p2, h1, inv, W1, b1r, g1, be1)
    (p3,) = sc_agg(h2, src, dst)
    (out,) = tc_last(p3, h2, inv, W2, b2r, g1, be1)
    return out


# speedup vs baseline: 1.8959x; 1.8959x over previous
import functools

import jax
import jax.numpy as jnp
from jax import lax
from jax.experimental import pallas as pl
from jax.experimental.pallas import tpu as pltpu
from jax.experimental.pallas import tpu_sc as plsc

N_NODES = 10000
N_EDGES = 320000
D = 128
EPS_BN = 1e-5

NC = 2
NS = 16
NW = NC * NS
EDGES_PER_WORKER = N_EDGES // NW
CHUNK = 80
NCHUNKS = EDGES_PER_WORKER // CHUNK
N_PAD = 10240
ROWS_PER_TILE = N_PAD // NS
ZROWS = 16


def _sc_agg_body(with_deg, *refs):
    if with_deg:
        (h_hbm, src_hbm, dst_hbm, out_hbm, deg_hbm,
         sidx_a, didx_a, sidx_b, didx_b, rows_a, rows_b, zeros_v,
         acc_sh, sem_a, sem_b) = refs
    else:
        (h_hbm, src_hbm, dst_hbm, out_hbm,
         sidx_a, didx_a, sidx_b, didx_b, rows_a, rows_b, zeros_v,
         acc_sh, sem_a, sem_b) = refs

    c = lax.axis_index("c")
    s = lax.axis_index("s")
    w = c * NS + s
    wbase = w * EDGES_PER_WORKER
    row0 = s * ROWS_PER_TILE

    zero16 = jnp.zeros((16,), jnp.float32)
    for r in range(ZROWS):
        for q in range(D // 16):
            zeros_v[r, pl.ds(q * 16, 16)] = zero16

    def zero_acc():
        for i in range(ROWS_PER_TILE // ZROWS):
            pltpu.sync_copy(zeros_v, acc_sh.at[pl.ds(row0 + i * ZROWS, ZROWS)])

    if with_deg:
        one16 = jnp.ones((16,), jnp.float32)
        for r in range(CHUNK):
            for q in range(D // 16):
                rows_a[r, pl.ds(q * 16, 16)] = one16
        zero_acc()
        plsc.subcore_barrier()

        @pl.loop(0, NCHUNKS)
        def _(j):
            base = pl.multiple_of(wbase + j * CHUNK, CHUNK)
            pltpu.sync_copy(dst_hbm.at[pl.ds(base, CHUNK)], didx_a)
            pltpu.sync_copy(rows_a, acc_sh.at[didx_a], add=True)

        plsc.subcore_barrier()
        pltpu.sync_copy(acc_sh.at[pl.ds(row0, ROWS_PER_TILE)],
                        deg_hbm.at[c, pl.ds(row0, ROWS_PER_TILE)])

    zero_acc()
    plsc.subcore_barrier()

    @pl.loop(0, NCHUNKS // 2)
    def _(i):
        base0 = pl.multiple_of(wbase + (2 * i) * CHUNK, CHUNK)
        base1 = pl.multiple_of(wbase + (2 * i + 1) * CHUNK, CHUNK)
        pltpu.sync_copy(src_hbm.at[pl.ds(base0, CHUNK)], sidx_a)
        pltpu.sync_copy(dst_hbm.at[pl.ds(base0, CHUNK)], didx_a)
        ga = pltpu.async_copy(h_hbm.at[sidx_a], rows_a, sem_a)
        pltpu.sync_copy(src_hbm.at[pl.ds(base1, CHUNK)], sidx_b)
        pltpu.sync_copy(dst_hbm.at[pl.ds(base1, CHUNK)], didx_b)
        gb = pltpu.async_copy(h_hbm.at[sidx_b], rows_b, sem_b)
        ga.wait()
        pltpu.sync_copy(rows_a, acc_sh.at[didx_a], add=True)
        gb.wait()
        pltpu.sync_copy(rows_b, acc_sh.at[didx_b], add=True)

    base = pl.multiple_of(wbase + (NCHUNKS - 1) * CHUNK, CHUNK)
    pltpu.sync_copy(src_hbm.at[pl.ds(base, CHUNK)], sidx_a)
    pltpu.sync_copy(dst_hbm.at[pl.ds(base, CHUNK)], didx_a)
    pltpu.async_copy(h_hbm.at[sidx_a], rows_a, sem_a).wait()
    pltpu.sync_copy(rows_a, acc_sh.at[didx_a], add=True)

    plsc.subcore_barrier()

    pltpu.sync_copy(acc_sh.at[pl.ds(row0, ROWS_PER_TILE)],
                    out_hbm.at[c, pl.ds(row0, ROWS_PER_TILE)])


def _make_sc_agg(with_deg):
    mesh = plsc.VectorSubcoreMesh(core_axis_name="c", subcore_axis_name="s",
                                  num_cores=NC, num_subcores=NS)
    out_type = [jax.ShapeDtypeStruct((NC, N_PAD, D), jnp.float32)]
    if with_deg:
        out_type.append(jax.ShapeDtypeStruct((NC, N_PAD, D), jnp.float32))
    scratch = [
        pltpu.VMEM((CHUNK,), jnp.int32),
        pltpu.VMEM((CHUNK,), jnp.int32),
        pltpu.VMEM((CHUNK,), jnp.int32),
        pltpu.VMEM((CHUNK,), jnp.int32),
        pltpu.VMEM((CHUNK, D), jnp.float32),
        pltpu.VMEM((CHUNK, D), jnp.float32),
        pltpu.VMEM((ZROWS, D), jnp.float32),
        pltpu.VMEM_SHARED((N_PAD, D), jnp.float32),
        pltpu.SemaphoreType.DMA,
        pltpu.SemaphoreType.DMA,
    ]
    return pl.kernel(
        functools.partial(_sc_agg_body, with_deg),
        out_type=out_type,
        mesh=mesh,
        scratch_types=scratch,
    )


def _tc_layer_body(first, bn, p_ref, h_ref, aux_ref, w_ref, b_ref,
                   gamma_ref, beta_ref, o_ref, inv_ref=None):
    if first:
        deg = aux_ref[0, :N_NODES, 0:1] + aux_ref[1, :N_NODES, 0:1]
        inv = 1.0 / jnp.maximum(deg, 1.0)
        inv_ref[...] = inv
    else:
        inv = aux_ref[...]
    p = p_ref[0, :N_NODES, :D] + p_ref[1, :N_NODES, :D]
    t = h_ref[...] + p * inv
    y = lax.dot_general(t, w_ref[...], (((1,), (1,)), ((), ())),
                        preferred_element_type=jnp.float32) + b_ref[...]
    if bn:
        mu = jnp.mean(y, axis=0, keepdims=True)
        var = jnp.mean((y - mu) * (y - mu), axis=0, keepdims=True)
        y = gamma_ref[...] * (y - mu) * lax.rsqrt(var + EPS_BN) + beta_ref[...]
        y = jnp.maximum(y, 0.0)
    o_ref[...] = y


def _make_tc_layer(first, bn):
    out_shape = [jax.ShapeDtypeStruct((N_NODES, D), jnp.float32)]
    if first:
        out_shape.append(jax.ShapeDtypeStruct((N_NODES, 1), jnp.float32))
    return pl.pallas_call(
        functools.partial(_tc_layer_body, first, bn),
        out_shape=out_shape,
    )


def kernel(x, edge_index, W0, b0, W1, b1, W2, b2,
           gamma0, beta0, gamma1, beta1):
    src = edge_index[0].astype(jnp.int32)
    dst = edge_index[1].astype(jnp.int32)

    sc_agg_deg = _make_sc_agg(True)
    sc_agg = _make_sc_agg(False)
    tc_first = _make_tc_layer(True, True)
    tc_mid = _make_tc_layer(False, True)
    tc_last = _make_tc_layer(False, False)

    b0r = b0.reshape(1, D)
    b1r = b1.reshape(1, D)
    b2r = b2.reshape(1, D)
    g0 = gamma0.reshape(1, D)
    g1 = gamma1.reshape(1, D)
    be0 = beta0.reshape(1, D)
    be1 = beta1.reshape(1, D)

    p1, degp = sc_agg_deg(x, src, dst)
    h1, inv = tc_first(p1, x, degp, W0, b0r, g0, be0)
    (p2,) = sc_agg(h1, src, dst)
    (h2,) = tc_mid(p2, h1, inv, W1, b1r, g1, be1)
    (p3,) = sc_agg(h2, src, dst)
    (out,) = tc_last(p3, h2, inv, W2, b2r, g1, be1)
    return out
